# baseline (speedup 1.0000x reference)
import functools

import jax
import jax.numpy as jnp
from jax import lax
from jax.experimental import pallas as pl
from jax.experimental.pallas import tpu as pltpu
from jax.experimental.pallas import tpu_sc as plsc

N = 10000
E = 320000
D = 128

NC = 2
NS = 16
NW = NC * NS

K = 128
NCHUNK = 80
E_PAD = NW * NCHUNK * K
N_PAD = 10240
STRIPE = N_PAD // NS

_mesh = plsc.VectorSubcoreMesh(core_axis_name="c", subcore_axis_name="s")


def _deg_body(src_hbm, dst_hbm, ones_hbm, zeros_hbm, out_hbm,
              sh_deg, ones_v, idxs_v, idxd_v, sem_a):
    c = lax.axis_index("c")
    s = lax.axis_index("s")
    wid = s * NC + c
    sl = pl.ds(s * STRIPE, STRIPE)

    pltpu.sync_copy(zeros_hbm.at[sl], sh_deg.at[sl])
    pltpu.sync_copy(ones_hbm, ones_v)
    pltpu.sync_copy(src_hbm.at[wid], idxs_v)
    pltpu.sync_copy(dst_hbm.at[wid], idxd_v)
    plsc.subcore_barrier()

    def pass_one(idx_v, out_slot):
        def body(j, carry):
            descs = [
                pltpu.async_copy(ones_v, sh_deg.at[idx_v.at[8 * j + t]],
                                 sem_a, add=True)
                for t in range(8)
            ]
            for desc in descs:
                desc.wait()
            return carry

        lax.fori_loop(0, NCHUNK // 8, body, 0)
        plsc.subcore_barrier()
        pltpu.sync_copy(sh_deg.at[sl], out_hbm.at[c, out_slot, sl])
        pltpu.sync_copy(zeros_hbm.at[sl], sh_deg.at[sl])
        plsc.subcore_barrier()

    pass_one(idxs_v, 0)
    pass_one(idxd_v, 1)


@jax.jit
def _deg_call(src2d, dst2d, ones, zerosND):
    return pl.kernel(
        _deg_body,
        out_type=jax.ShapeDtypeStruct((NC, 2, N_PAD, D), jnp.float32),
        mesh=_mesh,
        scratch_types=[
            pltpu.VMEM_SHARED((N_PAD, D), jnp.float32),
            pltpu.VMEM((K, D), jnp.float32),
            pltpu.VMEM((NCHUNK, K), jnp.int32),
            pltpu.VMEM((NCHUNK, K), jnp.int32),
            pltpu.SemaphoreType.DMA,
        ],
    )(src2d, dst2d, ones, zerosND)


def _agg_body(xs_hbm, src_hbm, dst_hbm, zeros_hbm, out_hbm,
              sh_acc, idxs_v, idxd_v, rows0, rows1, sem_g, sem_a):
    c = lax.axis_index("c")
    s = lax.axis_index("s")
    wid = s * NC + c
    sl = pl.ds(s * STRIPE, STRIPE)

    pltpu.sync_copy(zeros_hbm.at[sl], sh_acc.at[sl])
    plsc.subcore_barrier()

    rows = (rows0, rows1)

    def fire_gather(ch, buf):
        return pltpu.async_copy(xs_hbm.at[idxs_v.at[ch]], buf, sem_g)

    def wait_gather(ch, buf):
        pltpu.make_async_copy(xs_hbm.at[idxs_v.at[ch]], buf, sem_g).wait()

    HC = NCHUNK // 2
    for h in range(2):
        pltpu.sync_copy(src_hbm.at[wid, pl.ds(h * HC, HC)], idxs_v)
        pltpu.sync_copy(dst_hbm.at[wid, pl.ds(h * HC, HC)], idxd_v)

        fire_gather(0, rows0)
        fire_gather(1, rows1)

        def body(j, carry):
            adds = []
            for b in range(2):
                ch = 2 * j + b
                wait_gather(ch, rows[b])
                adds.append(pltpu.async_copy(rows[b],
                                             sh_acc.at[idxd_v.at[ch]],
                                             sem_a, add=True))
            for b in range(2):
                adds[b].wait()
                fire_gather(2 * j + 2 + b, rows[b])
            return carry

        lax.fori_loop(0, HC // 2 - 1, body, 0)

        adds = []
        for b in range(2):
            ch = HC - 2 + b
            wait_gather(ch, rows[b])
            adds.append(pltpu.async_copy(rows[b], sh_acc.at[idxd_v.at[ch]],
                                         sem_a, add=True))
        for add in adds:
            add.wait()
    plsc.subcore_barrier()

    pltpu.sync_copy(sh_acc.at[sl], out_hbm.at[c, sl])


@jax.jit
def _agg_call(xs, src2d, dst2d, zerosND):
    return pl.kernel(
        _agg_body,
        out_type=jax.ShapeDtypeStruct((NC, N_PAD, D), jnp.float32),
        mesh=_mesh,
        scratch_types=[
            pltpu.VMEM_SHARED((N_PAD, D), jnp.float32),
            pltpu.VMEM((NCHUNK // 2, K), jnp.int32),
            pltpu.VMEM((NCHUNK // 2, K), jnp.int32),
            pltpu.VMEM((K, D), jnp.float32),
            pltpu.VMEM((K, D), jnp.float32),
            pltpu.SemaphoreType.DMA,
            pltpu.SemaphoreType.DMA,
        ],
    )(xs, src2d, dst2d, zerosND)


BN = 1024


def _norms_body(deg_ref, out_ref):
    for slot in range(2):
        deg = deg_ref[0, slot, :, 0:1] + deg_ref[1, slot, :, 0:1]
        norm = lax.rsqrt(jnp.maximum(deg, 1.0))
        out_ref[slot] = jnp.broadcast_to(norm, (BN, 16))


@jax.jit
def _norms(degP):
    return pl.pallas_call(
        _norms_body,
        grid=(N_PAD // BN,),
        in_specs=[pl.BlockSpec((NC, 2, BN, D), lambda i: (0, 0, i, 0))],
        out_specs=pl.BlockSpec((2, BN, 16), lambda i: (0, i, 0)),
        out_shape=jax.ShapeDtypeStruct((2, N_PAD, 16), jnp.float32),
    )(degP)


def _prescale_body(x_ref, norm_ref, out_ref):
    out_ref[...] = x_ref[...] * norm_ref[0, :, 0:1]


@jax.jit
def _prescale(xpad, normP):
    return pl.pallas_call(
        _prescale_body,
        grid=(N_PAD // BN,),
        in_specs=[
            pl.BlockSpec((BN, D), lambda i: (i, 0)),
            pl.BlockSpec((2, BN, 16), lambda i: (0, i, 0)),
        ],
        out_specs=pl.BlockSpec((BN, D), lambda i: (i, 0)),
        out_shape=jax.ShapeDtypeStruct((N_PAD, D), jnp.float32),
    )(xpad, normP)


def _combine_body(agg_ref, norm_ref, w_ref, b_ref, out_ref, *, scale_out):
    a = agg_ref[0] + agg_ref[1]
    a = a * norm_ref[1, :, 0:1]
    h = jnp.dot(a, w_ref[...], preferred_element_type=jnp.float32)
    h = jnp.maximum(h + b_ref[...], 0.0)
    if scale_out:
        h = h * norm_ref[0, :, 0:1]
    out_ref[...] = h


@functools.partial(jax.jit, static_argnames=("scale_out",))
def _combine(aggP, normP, W, b2d, scale_out):
    return pl.pallas_call(
        functools.partial(_combine_body, scale_out=scale_out),
        grid=(N_PAD // BN,),
        in_specs=[
            pl.BlockSpec((NC, BN, D), lambda i: (0, i, 0)),
            pl.BlockSpec((2, BN, 16), lambda i: (0, i, 0)),
            pl.BlockSpec((D, D), lambda i: (0, 0)),
            pl.BlockSpec((1, D), lambda i: (0, 0)),
        ],
        out_specs=pl.BlockSpec((BN, D), lambda i: (i, 0)),
        out_shape=jax.ShapeDtypeStruct((N_PAD, D), jnp.float32),
    )(aggP, normP, W, b2d)


def kernel(x, edge_index, W1, b1, W2, b2):
    src = edge_index[0]
    dst = edge_index[1]
    pad = E_PAD - E
    fill = jnp.full((pad,), N, dtype=jnp.int32)
    src2d = jnp.concatenate([src, fill]).reshape(NW, NCHUNK, K)
    dst2d = jnp.concatenate([dst, fill]).reshape(NW, NCHUNK, K)
    xpad = jnp.pad(x, ((0, N_PAD - N), (0, 0)))

    ones = jnp.ones((K, D), dtype=jnp.float32)
    zerosND = jnp.zeros((N_PAD, D), dtype=jnp.float32)

    degP = _deg_call(src2d, dst2d, ones, zerosND)
    normP = _norms(degP)
    xs = _prescale(xpad, normP)
    agg1 = _agg_call(xs, src2d, dst2d, zerosND)
    h1s = _combine(agg1, normP, W1, b1.reshape(1, D), True)
    agg2 = _agg_call(h1s, src2d, dst2d, zerosND)
    h2 = _combine(agg2, normP, W2, b2.reshape(1, D), False)
    return h2[:N]


# speedup vs baseline: 1.0228x; 1.0228x over previous
import functools

import jax
import jax.numpy as jnp
from jax import lax
from jax.experimental import pallas as pl
from jax.experimental.pallas import tpu as pltpu
from jax.experimental.pallas import tpu_sc as plsc

N = 10000
E = 320000
D = 128

NC = 2
NS = 16
NW = NC * NS

K = 128
NCHUNK = 80
E_PAD = NW * NCHUNK * K
N_PAD = 10240
STRIPE = N_PAD // NS

_mesh = plsc.VectorSubcoreMesh(core_axis_name="c", subcore_axis_name="s")


def _deg_body(src_hbm, dst_hbm, ones_hbm, zeros_hbm, out_hbm,
              sh_deg, ones_v, idxs_v, idxd_v, sem_a):
    c = lax.axis_index("c")
    s = lax.axis_index("s")
    wid = s * NC + c
    sl = pl.ds(s * STRIPE, STRIPE)

    pltpu.sync_copy(zeros_hbm.at[sl], sh_deg.at[sl])
    pltpu.sync_copy(ones_hbm, ones_v)
    pltpu.sync_copy(src_hbm.at[wid], idxs_v)
    pltpu.sync_copy(dst_hbm.at[wid], idxd_v)
    plsc.subcore_barrier()

    def pass_one(idx_v, out_slot):
        def body(j, carry):
            descs = [
                pltpu.async_copy(ones_v, sh_deg.at[idx_v.at[8 * j + t]],
                                 sem_a, add=True)
                for t in range(8)
            ]
            for desc in descs:
                desc.wait()
            return carry

        lax.fori_loop(0, NCHUNK // 8, body, 0)
        plsc.subcore_barrier()
        pltpu.sync_copy(sh_deg.at[sl], out_hbm.at[c, out_slot, sl])
        pltpu.sync_copy(zeros_hbm.at[sl], sh_deg.at[sl])
        plsc.subcore_barrier()

    pass_one(idxs_v, 0)
    pass_one(idxd_v, 1)


@jax.jit
def _deg_call(src2d, dst2d, ones, zerosND):
    return pl.kernel(
        _deg_body,
        out_type=jax.ShapeDtypeStruct((NC, 2, N_PAD, D), jnp.float32),
        mesh=_mesh,
        scratch_types=[
            pltpu.VMEM_SHARED((N_PAD, D), jnp.float32),
            pltpu.VMEM((K, D), jnp.float32),
            pltpu.VMEM((NCHUNK, K), jnp.int32),
            pltpu.VMEM((NCHUNK, K), jnp.int32),
            pltpu.SemaphoreType.DMA,
        ],
    )(src2d, dst2d, ones, zerosND)


def _agg_body(xs_hbm, src_hbm, dst_hbm, zeros_hbm, out_hbm,
              sh_acc, idxs_v, idxd_v, rows0, rows1, sem_g, sem_a):
    c = lax.axis_index("c")
    s = lax.axis_index("s")
    wid = s * NC + c
    sl = pl.ds(s * STRIPE, STRIPE)

    pltpu.sync_copy(zeros_hbm.at[sl], sh_acc.at[sl])
    plsc.subcore_barrier()

    rows = (rows0, rows1)

    def fire_gather(ch, buf):
        return pltpu.async_copy(xs_hbm.at[idxs_v.at[ch]], buf, sem_g)

    def wait_gather(ch, buf):
        pltpu.make_async_copy(xs_hbm.at[idxs_v.at[ch]], buf, sem_g).wait()

    HC = NCHUNK // 2
    for h in range(2):
        pltpu.sync_copy(src_hbm.at[wid, pl.ds(h * HC, HC)], idxs_v)
        pltpu.sync_copy(dst_hbm.at[wid, pl.ds(h * HC, HC)], idxd_v)

        fire_gather(0, rows0)
        fire_gather(1, rows1)

        def body(j, carry):
            for b in range(2):
                ch = 2 * j + b
                wait_gather(ch, rows[b])
                pltpu.sync_copy(rows[b], sh_acc.at[idxd_v.at[ch]], add=True)
                fire_gather(2 * j + 2 + b, rows[b])
            return carry

        lax.fori_loop(0, HC // 2 - 1, body, 0)

        for b in range(2):
            ch = HC - 2 + b
            wait_gather(ch, rows[b])
            pltpu.sync_copy(rows[b], sh_acc.at[idxd_v.at[ch]], add=True)
    plsc.subcore_barrier()

    pltpu.sync_copy(sh_acc.at[sl], out_hbm.at[c, sl])


@jax.jit
def _agg_call(xs, src2d, dst2d, zerosND):
    return pl.kernel(
        _agg_body,
        out_type=jax.ShapeDtypeStruct((NC, N_PAD, D), jnp.float32),
        mesh=_mesh,
        scratch_types=[
            pltpu.VMEM_SHARED((N_PAD, D), jnp.float32),
            pltpu.VMEM((NCHUNK // 2, K), jnp.int32),
            pltpu.VMEM((NCHUNK // 2, K), jnp.int32),
            pltpu.VMEM((K, D), jnp.float32),
            pltpu.VMEM((K, D), jnp.float32),
            pltpu.SemaphoreType.DMA,
            pltpu.SemaphoreType.DMA,
        ],
    )(xs, src2d, dst2d, zerosND)


BN = 1024


def _norms_body(deg_ref, out_ref):
    for slot in range(2):
        deg = deg_ref[0, slot, :, 0:1] + deg_ref[1, slot, :, 0:1]
        norm = lax.rsqrt(jnp.maximum(deg, 1.0))
        out_ref[slot] = jnp.broadcast_to(norm, (BN, 16))


@jax.jit
def _norms(degP):
    return pl.pallas_call(
        _norms_body,
        grid=(N_PAD // BN,),
        in_specs=[pl.BlockSpec((NC, 2, BN, D), lambda i: (0, 0, i, 0))],
        out_specs=pl.BlockSpec((2, BN, 16), lambda i: (0, i, 0)),
        out_shape=jax.ShapeDtypeStruct((2, N_PAD, 16), jnp.float32),
    )(degP)


def _prescale_body(x_ref, norm_ref, out_ref):
    out_ref[...] = x_ref[...] * norm_ref[0, :, 0:1]


@jax.jit
def _prescale(xpad, normP):
    return pl.pallas_call(
        _prescale_body,
        grid=(N_PAD // BN,),
        in_specs=[
            pl.BlockSpec((BN, D), lambda i: (i, 0)),
            pl.BlockSpec((2, BN, 16), lambda i: (0, i, 0)),
        ],
        out_specs=pl.BlockSpec((BN, D), lambda i: (i, 0)),
        out_shape=jax.ShapeDtypeStruct((N_PAD, D), jnp.float32),
    )(xpad, normP)


def _combine_body(agg_ref, norm_ref, w_ref, b_ref, out_ref, *, scale_out):
    a = agg_ref[0] + agg_ref[1]
    a = a * norm_ref[1, :, 0:1]
    h = jnp.dot(a, w_ref[...], preferred_element_type=jnp.float32)
    h = jnp.maximum(h + b_ref[...], 0.0)
    if scale_out:
        h = h * norm_ref[0, :, 0:1]
    out_ref[...] = h


@functools.partial(jax.jit, static_argnames=("scale_out",))
def _combine(aggP, normP, W, b2d, scale_out):
    return pl.pallas_call(
        functools.partial(_combine_body, scale_out=scale_out),
        grid=(N_PAD // BN,),
        in_specs=[
            pl.BlockSpec((NC, BN, D), lambda i: (0, i, 0)),
            pl.BlockSpec((2, BN, 16), lambda i: (0, i, 0)),
            pl.BlockSpec((D, D), lambda i: (0, 0)),
            pl.BlockSpec((1, D), lambda i: (0, 0)),
        ],
        out_specs=pl.BlockSpec((BN, D), lambda i: (i, 0)),
        out_shape=jax.ShapeDtypeStruct((N_PAD, D), jnp.float32),
    )(aggP, normP, W, b2d)


def kernel(x, edge_index, W1, b1, W2, b2):
    src = edge_index[0]
    dst = edge_index[1]
    pad = E_PAD - E
    fill = jnp.full((pad,), N, dtype=jnp.int32)
    src2d = jnp.concatenate([src, fill]).reshape(NW, NCHUNK, K)
    dst2d = jnp.concatenate([dst, fill]).reshape(NW, NCHUNK, K)
    xpad = jnp.pad(x, ((0, N_PAD - N), (0, 0)))

    ones = jnp.ones((K, D), dtype=jnp.float32)
    zerosND = jnp.zeros((N_PAD, D), dtype=jnp.float32)

    degP = _deg_call(src2d, dst2d, ones, zerosND)
    normP = _norms(degP)
    xs = _prescale(xpad, normP)
    agg1 = _agg_call(xs, src2d, dst2d, zerosND)
    h1s = _combine(agg1, normP, W1, b1.reshape(1, D), True)
    agg2 = _agg_call(h1s, src2d, dst2d, zerosND)
    h2 = _combine(agg2, normP, W2, b2.reshape(1, D), False)
    return h2[:N]


# speedup vs baseline: 1.2521x; 1.2242x over previous
import functools

import jax
import jax.numpy as jnp
from jax import lax
from jax.experimental import pallas as pl
from jax.experimental.pallas import tpu as pltpu
from jax.experimental.pallas import tpu_sc as plsc

N = 10000
E = 320000
D = 128

NC = 2
NS = 16
NW = NC * NS

K = 128
NCHUNK = -(-E // (NW * K))
E_PAD = NW * NCHUNK * K
N_PAD = 10240
STRIPE = N_PAD // NS

_mesh = plsc.VectorSubcoreMesh(core_axis_name="c", subcore_axis_name="s")


def _deg_body(src_hbm, dst_hbm, ones_hbm, zeros_hbm, out_hbm,
              sh_deg, ones_v, idxs_v, idxd_v):
    c = lax.axis_index("c")
    s = lax.axis_index("s")
    wid = s * NC + c
    sl = pl.ds(s * STRIPE, STRIPE)

    pltpu.sync_copy(zeros_hbm.at[sl], sh_deg.at[sl])
    pltpu.sync_copy(ones_hbm, ones_v)
    pltpu.sync_copy(src_hbm.at[wid], idxs_v)
    pltpu.sync_copy(dst_hbm.at[wid], idxd_v)
    plsc.subcore_barrier()

    def pass_one(idx_v, out_slot):
        def body(ch, carry):
            pltpu.sync_copy(ones_v, sh_deg.at[idx_v.at[ch]], add=True)
            return carry

        lax.fori_loop(0, NCHUNK, body, 0)
        plsc.subcore_barrier()
        pltpu.sync_copy(sh_deg.at[sl], out_hbm.at[c, out_slot, sl])
        pltpu.sync_copy(zeros_hbm.at[sl], sh_deg.at[sl])
        plsc.subcore_barrier()

    pass_one(idxs_v, 0)
    pass_one(idxd_v, 1)


@jax.jit
def _deg_call(src2d, dst2d, ones, zerosND):
    return pl.kernel(
        _deg_body,
        out_type=jax.ShapeDtypeStruct((NC, 2, N_PAD, D), jnp.float32),
        mesh=_mesh,
        scratch_types=[
            pltpu.VMEM_SHARED((N_PAD, D), jnp.float32),
            pltpu.VMEM((K, D), jnp.float32),
            pltpu.VMEM((NCHUNK, K), jnp.int32),
            pltpu.VMEM((NCHUNK, K), jnp.int32),
        ],
    )(src2d, dst2d, ones, zerosND)


def _agg_body(xs_hbm, src_hbm, dst_hbm, zeros_hbm, out_hbm,
              sh_acc, idxs_v, idxd_v, rows_v, sem):
    c = lax.axis_index("c")
    s = lax.axis_index("s")
    wid = s * NC + c
    sl = pl.ds(s * STRIPE, STRIPE)

    pltpu.sync_copy(zeros_hbm.at[sl], sh_acc.at[sl])
    pltpu.sync_copy(src_hbm.at[wid], idxs_v)
    pltpu.sync_copy(dst_hbm.at[wid], idxd_v)
    plsc.subcore_barrier()

    def body(ch, carry):
        pltpu.async_copy(xs_hbm.at[idxs_v.at[ch]], rows_v, sem).wait()
        pltpu.sync_copy(rows_v, sh_acc.at[idxd_v.at[ch]], add=True)
        return carry

    lax.fori_loop(0, NCHUNK, body, 0)
    plsc.subcore_barrier()

    pltpu.sync_copy(sh_acc.at[sl], out_hbm.at[c, sl])


@jax.jit
def _agg_call(xs, src2d, dst2d, zerosND):
    return pl.kernel(
        _agg_body,
        out_type=jax.ShapeDtypeStruct((NC, N_PAD, D), jnp.float32),
        mesh=_mesh,
        scratch_types=[
            pltpu.VMEM_SHARED((N_PAD, D), jnp.float32),
            pltpu.VMEM((NCHUNK, K), jnp.int32),
            pltpu.VMEM((NCHUNK, K), jnp.int32),
            pltpu.VMEM((K, D), jnp.float32),
            pltpu.SemaphoreType.DMA,
        ],
    )(xs, src2d, dst2d, zerosND)


BN = 1024


def _norms_body(deg_ref, out_ref):
    for slot in range(2):
        deg = deg_ref[0, slot, :, 0:1] + deg_ref[1, slot, :, 0:1]
        norm = lax.rsqrt(jnp.maximum(deg, 1.0))
        out_ref[slot] = jnp.broadcast_to(norm, (BN, 16))


@jax.jit
def _norms(degP):
    return pl.pallas_call(
        _norms_body,
        grid=(N_PAD // BN,),
        in_specs=[pl.BlockSpec((NC, 2, BN, D), lambda i: (0, 0, i, 0))],
        out_specs=pl.BlockSpec((2, BN, 16), lambda i: (0, i, 0)),
        out_shape=jax.ShapeDtypeStruct((2, N_PAD, 16), jnp.float32),
    )(degP)


def _prescale_body(x_ref, norm_ref, out_ref):
    out_ref[...] = x_ref[...] * norm_ref[0, :, 0:1]


@jax.jit
def _prescale(xpad, normP):
    return pl.pallas_call(
        _prescale_body,
        grid=(N_PAD // BN,),
        in_specs=[
            pl.BlockSpec((BN, D), lambda i: (i, 0)),
            pl.BlockSpec((2, BN, 16), lambda i: (0, i, 0)),
        ],
        out_specs=pl.BlockSpec((BN, D), lambda i: (i, 0)),
        out_shape=jax.ShapeDtypeStruct((N_PAD, D), jnp.float32),
    )(xpad, normP)


def _combine_body(agg_ref, norm_ref, w_ref, b_ref, out_ref, *, scale_out):
    a = agg_ref[0] + agg_ref[1]
    a = a * norm_ref[1, :, 0:1]
    h = jnp.dot(a, w_ref[...], preferred_element_type=jnp.float32)
    h = jnp.maximum(h + b_ref[...], 0.0)
    if scale_out:
        h = h * norm_ref[0, :, 0:1]
    out_ref[...] = h


@functools.partial(jax.jit, static_argnames=("scale_out",))
def _combine(aggP, normP, W, b2d, scale_out):
    return pl.pallas_call(
        functools.partial(_combine_body, scale_out=scale_out),
        grid=(N_PAD // BN,),
        in_specs=[
            pl.BlockSpec((NC, BN, D), lambda i: (0, i, 0)),
            pl.BlockSpec((2, BN, 16), lambda i: (0, i, 0)),
            pl.BlockSpec((D, D), lambda i: (0, 0)),
            pl.BlockSpec((1, D), lambda i: (0, 0)),
        ],
        out_specs=pl.BlockSpec((BN, D), lambda i: (i, 0)),
        out_shape=jax.ShapeDtypeStruct((N_PAD, D), jnp.float32),
    )(aggP, normP, W, b2d)


def kernel(x, edge_index, W1, b1, W2, b2):
    src = edge_index[0]
    dst = edge_index[1]
    pad = E_PAD - E
    fill = jnp.full((pad,), N, dtype=jnp.int32)
    src2d = jnp.concatenate([src, fill]).reshape(NW, NCHUNK, K)
    dst2d = jnp.concatenate([dst, fill]).reshape(NW, NCHUNK, K)
    xpad = jnp.pad(x, ((0, N_PAD - N), (0, 0)))

    ones = jnp.ones((K, D), dtype=jnp.float32)
    zerosND = jnp.zeros((N_PAD, D), dtype=jnp.float32)

    degP = _deg_call(src2d, dst2d, ones, zerosND)
    normP = _norms(degP)
    xs = _prescale(xpad, normP)
    agg1 = _agg_call(xs, src2d, dst2d, zerosND)
    h1s = _combine(agg1, normP, W1, b1.reshape(1, D), True)
    agg2 = _agg_call(h1s, src2d, dst2d, zerosND)
    h2 = _combine(agg2, normP, W2, b2.reshape(1, D), False)
    return h2[:N]
